# Initial kernel scaffold; baseline (speedup 1.0000x reference)
import jax
import jax.numpy as jnp
from jax.experimental import pallas as pl


def kernel(edge_index, edge_weight, user_emb, item_emb):
    raise NotImplementedError("write your pallas kernel here")


# speedup vs baseline: 5.1688x; 5.1688x over previous
import functools

import jax
import jax.numpy as jnp
from jax import lax
from jax.experimental import pallas as pl
from jax.experimental.pallas import tpu as pltpu
from jax.experimental.pallas import tpu_sc as plsc

_NUM_USERS = 25000
_NUM_ITEMS = 25000
_N = _NUM_USERS + _NUM_ITEMS
_NPAD = 50176
_D = 64
_HALF = 32
_E = 800000
_LAYERS = 3

_NC = 2
_NS = 16
_LANES = 16

_CHUNK = 80
_NCHUNKS = _E // _CHUNK
_CPT = _NCHUNKS // _NS
_RPT = _NPAD // _NS
_RCHUNK = 112
_RSTEPS = _RPT // _RCHUNK


def _gcn_body(src_hbm, dst_hbm, w_hbm, tbl_hbm, out_hbm, lay_hbm,
              acc, idx_src0, idx_src1, idx_dst0, idx_dst1, w0, w1,
              rows0, rows1, zbuf, b0, b1, b2, b3,
              sem_i0, sem_i1, sem_g0, sem_g1):
  c = lax.axis_index("c")
  s = lax.axis_index("s")
  row_base = s * _RPT

  idx_src = (idx_src0, idx_src1)
  idx_dst = (idx_dst0, idx_dst1)
  wv = (w0, w1)
  rows = (rows0, rows1)
  sem_i = (sem_i0, sem_i1)
  sem_g = (sem_g0, sem_g1)

  def _zb(i, _):
    zbuf[i, pl.ds(0, 16)] = jnp.zeros((16,), jnp.float32)
    zbuf[i, pl.ds(16, 16)] = jnp.zeros((16,), jnp.float32)
    return 0
  lax.fori_loop(0, _RCHUNK, _zb, 0, unroll=4)

  def zero_acc():
    def body(z, _):
      pltpu.sync_copy(zbuf, acc.at[pl.ds(row_base + z * _RCHUNK, _RCHUNK)])
      return 0
    lax.fori_loop(0, _RSTEPS, body, 0)

  def fetch_idx(j, b):
    r = s * _CPT + j
    pltpu.async_copy(src_hbm.at[r, 0], idx_src[b], sem_i[b])
    pltpu.async_copy(dst_hbm.at[r, 0], idx_dst[b], sem_i[b])
    pltpu.async_copy(w_hbm.at[r, 0], wv[b], sem_i[b])

  def start_gather(table, b):
    pltpu.make_async_copy(src_hbm.at[0, 0], idx_src[b], sem_i[b]).wait()
    pltpu.make_async_copy(dst_hbm.at[0, 0], idx_dst[b], sem_i[b]).wait()
    pltpu.make_async_copy(w_hbm.at[0, 0], wv[b], sem_i[b]).wait()
    pltpu.async_copy(table.at[idx_src[b]], rows[b], sem_g[b])

  def finish_chunk(table, b):
    pltpu.make_async_copy(table.at[idx_src[b]], rows[b], sem_g[b]).wait()

    def scale(g, _):
      wvec = wv[b][pl.ds(g * _LANES, _LANES)]
      base = g * _LANES
      for t in range(_LANES):
        we = jnp.full((_LANES,), wvec[t])
        e = base + t
        rows[b][e, pl.ds(0, 16)] = rows[b][e, pl.ds(0, 16)] * we
        rows[b][e, pl.ds(16, 16)] = rows[b][e, pl.ds(16, 16)] * we
      return 0
    lax.fori_loop(0, _CHUNK // _LANES, scale, 0)

    pltpu.sync_copy(rows[b], acc.at[idx_dst[b]], add=True)

  def run_layer(table):
    fetch_idx(0, 0)
    start_gather(table, 0)
    fetch_idx(1, 1)

    def pair(k, _):
      j = 2 * k
      start_gather(table, 1)
      finish_chunk(table, 0)
      fetch_idx(j + 2, 0)
      start_gather(table, 0)
      finish_chunk(table, 1)
      @pl.when(j + 3 < _CPT)
      def _():
        fetch_idx(j + 3, 1)
      return 0

    lax.fori_loop(0, _CPT // 2, pair, 0)
    finish_chunk(table, 0)

  def writeback(dst_table):
    def body(z, _):
      r0 = row_base + z * _RCHUNK
      pltpu.sync_copy(acc.at[pl.ds(r0, _RCHUNK)],
                      dst_table.at[pl.ds(r0, _RCHUNK)])
      return 0
    lax.fori_loop(0, _RSTEPS, body, 0)

  zero_acc()
  plsc.subcore_barrier()
  run_layer(tbl_hbm.at[c])
  plsc.subcore_barrier()
  writeback(lay_hbm.at[0, c])
  plsc.subcore_barrier()

  zero_acc()
  plsc.subcore_barrier()
  run_layer(lay_hbm.at[0, c])
  plsc.subcore_barrier()
  writeback(lay_hbm.at[1, c])
  plsc.subcore_barrier()

  zero_acc()
  plsc.subcore_barrier()
  run_layer(lay_hbm.at[1, c])
  plsc.subcore_barrier()

  def mean_body(z, _):
    r0 = row_base + z * _RCHUNK
    pltpu.sync_copy(tbl_hbm.at[c, pl.ds(r0, _RCHUNK)], b0)
    pltpu.sync_copy(lay_hbm.at[0, c, pl.ds(r0, _RCHUNK)], b1)
    pltpu.sync_copy(lay_hbm.at[1, c, pl.ds(r0, _RCHUNK)], b2)
    pltpu.sync_copy(acc.at[pl.ds(r0, _RCHUNK)], b3)

    def avg(i, _):
      for h in (0, 16):
        m = (b0[i, pl.ds(h, 16)] + b1[i, pl.ds(h, 16)]
             + b2[i, pl.ds(h, 16)] + b3[i, pl.ds(h, 16)]) * 0.25
        b0[i, pl.ds(h, 16)] = m
      return 0
    lax.fori_loop(0, _RCHUNK, avg, 0, unroll=4)

    pltpu.sync_copy(b0, out_hbm.at[c, pl.ds(r0, _RCHUNK)])
    return 0
  lax.fori_loop(0, _RSTEPS, mean_body, 0)


@jax.jit
def _gcn(src2d, dst2d, w2d, tbl):
  mesh = plsc.VectorSubcoreMesh(core_axis_name="c", subcore_axis_name="s",
                                num_cores=_NC, num_subcores=_NS)
  f = pl.kernel(
      _gcn_body,
      out_type=(
          jax.ShapeDtypeStruct((_NC, _NPAD, _HALF), jnp.float32),
          jax.ShapeDtypeStruct((2, _NC, _NPAD, _HALF), jnp.float32),
      ),
      mesh=mesh,
      compiler_params=pltpu.CompilerParams(use_tc_tiling_on_sc=False),
      scratch_types=[
          pltpu.VMEM_SHARED((_NPAD, _HALF), jnp.float32),
          pltpu.VMEM((_CHUNK,), jnp.int32),
          pltpu.VMEM((_CHUNK,), jnp.int32),
          pltpu.VMEM((_CHUNK,), jnp.int32),
          pltpu.VMEM((_CHUNK,), jnp.int32),
          pltpu.VMEM((_CHUNK,), jnp.float32),
          pltpu.VMEM((_CHUNK,), jnp.float32),
          pltpu.VMEM((_CHUNK, _HALF), jnp.float32),
          pltpu.VMEM((_CHUNK, _HALF), jnp.float32),
          pltpu.VMEM((_RCHUNK, _HALF), jnp.float32),
          pltpu.VMEM((_RCHUNK, _HALF), jnp.float32),
          pltpu.VMEM((_RCHUNK, _HALF), jnp.float32),
          pltpu.VMEM((_RCHUNK, _HALF), jnp.float32),
          pltpu.VMEM((_RCHUNK, _HALF), jnp.float32),
          pltpu.SemaphoreType.DMA,
          pltpu.SemaphoreType.DMA,
          pltpu.SemaphoreType.DMA,
          pltpu.SemaphoreType.DMA,
      ],
  )
  out, _ = f(src2d, dst2d, w2d, tbl)
  return out


def kernel(edge_index, edge_weight, user_emb, item_emb):
  src3d = edge_index[0].reshape(_NCHUNKS, 1, _CHUNK)
  dst3d = edge_index[1].reshape(_NCHUNKS, 1, _CHUNK)
  w3d = edge_weight.reshape(_NCHUNKS, 1, _CHUNK)
  all_emb = jnp.concatenate([user_emb, item_emb], axis=0)
  pad = jnp.zeros((_NPAD - _N, _D), jnp.float32)
  all_emb = jnp.concatenate([all_emb, pad], axis=0)
  tbl = jnp.stack([all_emb[:, :_HALF], all_emb[:, _HALF:]], axis=0)
  out = _gcn(src3d, dst3d, w3d, tbl)
  final = jnp.concatenate([out[0, :_N], out[1, :_N]], axis=1)
  return final[:_NUM_USERS], final[_NUM_USERS:]


# speedup vs baseline: 7.1818x; 1.3894x over previous
import jax
import jax.numpy as jnp
from jax import lax
from jax.experimental import pallas as pl
from jax.experimental.pallas import tpu as pltpu
from jax.experimental.pallas import tpu_sc as plsc

_NUM_USERS = 25000
_NUM_ITEMS = 25000
_N = _NUM_USERS + _NUM_ITEMS
_NPAD = 50176
_D = 64
_HALF = 32
_E = 800000

_NC = 2
_NS = 16
_LANES = 16

_CHUNK = 80
_NCHUNKS = _E // _CHUNK
_CPT = _NCHUNKS // _NS
_NBUF = 4
_RPT = _NPAD // _NS
_RCHUNK = 112
_RSTEPS = _RPT // _RCHUNK


def _gcn_body(src_hbm, dst_hbm, w_hbm, tbl_hbm, out_hbm, lay_hbm,
              acc,
              is0, is1, is2, is3, id0, id1, id2, id3, w0, w1, w2, w3,
              r0_, r1_, r2_, r3_, zbuf, b0, b1, b2, b3,
              si0, si1, si2, si3, sg0, sg1, sg2, sg3, ss0, ss1, ss2, ss3):
  c = lax.axis_index("c")
  s = lax.axis_index("s")
  row_base = s * _RPT

  idx_src = (is0, is1, is2, is3)
  idx_dst = (id0, id1, id2, id3)
  wv = (w0, w1, w2, w3)
  rows = (r0_, r1_, r2_, r3_)
  sem_i = (si0, si1, si2, si3)
  sem_g = (sg0, sg1, sg2, sg3)
  sem_s = (ss0, ss1, ss2, ss3)

  def _zb(i, _):
    zbuf[i, pl.ds(0, 16)] = jnp.zeros((16,), jnp.float32)
    zbuf[i, pl.ds(16, 16)] = jnp.zeros((16,), jnp.float32)
    return 0
  lax.fori_loop(0, _RCHUNK, _zb, 0, unroll=4)

  def zero_acc():
    def body(z, _):
      pltpu.sync_copy(zbuf, acc.at[pl.ds(row_base + z * _RCHUNK, _RCHUNK)])
      return 0
    lax.fori_loop(0, _RSTEPS, body, 0)

  def fetch_idx(j, b):
    r = s * _CPT + j
    pltpu.async_copy(src_hbm.at[r, 0], idx_src[b], sem_i[b])
    pltpu.async_copy(dst_hbm.at[r, 0], idx_dst[b], sem_i[b])
    pltpu.async_copy(w_hbm.at[r, 0], wv[b], sem_i[b])

  def start_gather(table, b):
    pltpu.make_async_copy(src_hbm.at[0, 0], idx_src[b], sem_i[b]).wait()
    pltpu.make_async_copy(dst_hbm.at[0, 0], idx_dst[b], sem_i[b]).wait()
    pltpu.make_async_copy(w_hbm.at[0, 0], wv[b], sem_i[b]).wait()
    pltpu.async_copy(table.at[idx_src[b]], rows[b], sem_g[b])

  def wait_scatter(b):
    pltpu.make_async_copy(rows[b], acc.at[idx_dst[b]], sem_s[b]).wait()

  def process(table, b):
    pltpu.make_async_copy(table.at[idx_src[b]], rows[b], sem_g[b]).wait()

    def scale(g, _):
      wvec = wv[b][pl.ds(g * _LANES, _LANES)]
      base = g * _LANES
      for t in range(_LANES):
        we = jnp.full((_LANES,), wvec[t])
        e = base + t
        rows[b][e, pl.ds(0, 16)] = rows[b][e, pl.ds(0, 16)] * we
        rows[b][e, pl.ds(16, 16)] = rows[b][e, pl.ds(16, 16)] * we
      return 0
    lax.fori_loop(0, _CHUNK // _LANES, scale, 0)

    pltpu.async_copy(rows[b], acc.at[idx_dst[b]], sem_s[b], add=True)

  def step(table, j, slot):
    b_cur = slot
    b_nxt = (slot + 1) % _NBUF
    b_n2 = (slot + 2) % _NBUF

    @pl.when(j >= 2)
    def _():
      wait_scatter(b_n2)
    @pl.when(j + 2 < _CPT)
    def _():
      fetch_idx(j + 2, b_n2)
    @pl.when(j + 1 < _CPT)
    def _():
      start_gather(table, b_nxt)
    process(table, b_cur)

  def run_layer(table):
    fetch_idx(0, 0)
    start_gather(table, 0)
    fetch_idx(1, 1)

    def quad(k, _):
      for slot in range(_NBUF):
        step(table, _NBUF * k + slot, slot)
      return 0
    lax.fori_loop(0, _CPT // _NBUF, quad, 0)

    step(table, _CPT - 1, 0)
    wait_scatter(3)
    wait_scatter(0)

  def writeback(dst_table):
    def body(z, _):
      rr = row_base + z * _RCHUNK
      pltpu.sync_copy(acc.at[pl.ds(rr, _RCHUNK)],
                      dst_table.at[pl.ds(rr, _RCHUNK)])
      return 0
    lax.fori_loop(0, _RSTEPS, body, 0)

  zero_acc()
  plsc.subcore_barrier()
  run_layer(tbl_hbm.at[c])
  plsc.subcore_barrier()
  writeback(lay_hbm.at[0, c])
  plsc.subcore_barrier()

  zero_acc()
  plsc.subcore_barrier()
  run_layer(lay_hbm.at[0, c])
  plsc.subcore_barrier()
  writeback(lay_hbm.at[1, c])
  plsc.subcore_barrier()

  zero_acc()
  plsc.subcore_barrier()
  run_layer(lay_hbm.at[1, c])
  plsc.subcore_barrier()

  def mean_body(z, _):
    rr = row_base + z * _RCHUNK
    pltpu.sync_copy(tbl_hbm.at[c, pl.ds(rr, _RCHUNK)], b0)
    pltpu.sync_copy(lay_hbm.at[0, c, pl.ds(rr, _RCHUNK)], b1)
    pltpu.sync_copy(lay_hbm.at[1, c, pl.ds(rr, _RCHUNK)], b2)
    pltpu.sync_copy(acc.at[pl.ds(rr, _RCHUNK)], b3)

    def avg(i, _):
      for h in (0, 16):
        m = (b0[i, pl.ds(h, 16)] + b1[i, pl.ds(h, 16)]
             + b2[i, pl.ds(h, 16)] + b3[i, pl.ds(h, 16)]) * 0.25
        b0[i, pl.ds(h, 16)] = m
      return 0
    lax.fori_loop(0, _RCHUNK, avg, 0, unroll=4)

    pltpu.sync_copy(b0, out_hbm.at[c, pl.ds(rr, _RCHUNK)])
    return 0
  lax.fori_loop(0, _RSTEPS, mean_body, 0)


@jax.jit
def _gcn(src3d, dst3d, w3d, tbl):
  mesh = plsc.VectorSubcoreMesh(core_axis_name="c", subcore_axis_name="s",
                                num_cores=_NC, num_subcores=_NS)
  f = pl.kernel(
      _gcn_body,
      out_type=(
          jax.ShapeDtypeStruct((_NC, _NPAD, _HALF), jnp.float32),
          jax.ShapeDtypeStruct((2, _NC, _NPAD, _HALF), jnp.float32),
      ),
      mesh=mesh,
      compiler_params=pltpu.CompilerParams(use_tc_tiling_on_sc=False),
      scratch_types=[
          pltpu.VMEM_SHARED((_NPAD, _HALF), jnp.float32),
          pltpu.VMEM((_CHUNK,), jnp.int32),
          pltpu.VMEM((_CHUNK,), jnp.int32),
          pltpu.VMEM((_CHUNK,), jnp.int32),
          pltpu.VMEM((_CHUNK,), jnp.int32),
          pltpu.VMEM((_CHUNK,), jnp.int32),
          pltpu.VMEM((_CHUNK,), jnp.int32),
          pltpu.VMEM((_CHUNK,), jnp.int32),
          pltpu.VMEM((_CHUNK,), jnp.int32),
          pltpu.VMEM((_CHUNK,), jnp.float32),
          pltpu.VMEM((_CHUNK,), jnp.float32),
          pltpu.VMEM((_CHUNK,), jnp.float32),
          pltpu.VMEM((_CHUNK,), jnp.float32),
          pltpu.VMEM((_CHUNK, _HALF), jnp.float32),
          pltpu.VMEM((_CHUNK, _HALF), jnp.float32),
          pltpu.VMEM((_CHUNK, _HALF), jnp.float32),
          pltpu.VMEM((_CHUNK, _HALF), jnp.float32),
          pltpu.VMEM((_RCHUNK, _HALF), jnp.float32),
          pltpu.VMEM((_RCHUNK, _HALF), jnp.float32),
          pltpu.VMEM((_RCHUNK, _HALF), jnp.float32),
          pltpu.VMEM((_RCHUNK, _HALF), jnp.float32),
          pltpu.VMEM((_RCHUNK, _HALF), jnp.float32),
          pltpu.SemaphoreType.DMA,
          pltpu.SemaphoreType.DMA,
          pltpu.SemaphoreType.DMA,
          pltpu.SemaphoreType.DMA,
          pltpu.SemaphoreType.DMA,
          pltpu.SemaphoreType.DMA,
          pltpu.SemaphoreType.DMA,
          pltpu.SemaphoreType.DMA,
          pltpu.SemaphoreType.DMA,
          pltpu.SemaphoreType.DMA,
          pltpu.SemaphoreType.DMA,
          pltpu.SemaphoreType.DMA,
      ],
  )
  out, _ = f(src3d, dst3d, w3d, tbl)
  return out


def kernel(edge_index, edge_weight, user_emb, item_emb):
  src3d = edge_index[0].reshape(_NCHUNKS, 1, _CHUNK)
  dst3d = edge_index[1].reshape(_NCHUNKS, 1, _CHUNK)
  w3d = edge_weight.reshape(_NCHUNKS, 1, _CHUNK)
  all_emb = jnp.concatenate([user_emb, item_emb], axis=0)
  pad = jnp.zeros((_NPAD - _N, _D), jnp.float32)
  all_emb = jnp.concatenate([all_emb, pad], axis=0)
  tbl = jnp.stack([all_emb[:, :_HALF], all_emb[:, _HALF:]], axis=0)
  out = _gcn(src3d, dst3d, w3d, tbl)
  final = jnp.concatenate([out[0, :_N], out[1, :_N]], axis=1)
  return final[:_NUM_USERS], final[_NUM_USERS:]


# speedup vs baseline: 8.4166x; 1.1719x over previous
import jax
import jax.numpy as jnp
from jax import lax
from jax.experimental import pallas as pl
from jax.experimental.pallas import tpu as pltpu
from jax.experimental.pallas import tpu_sc as plsc

_NUM_USERS = 25000
_NUM_ITEMS = 25000
_N = _NUM_USERS + _NUM_ITEMS
_NPAD = 50176
_D = 64
_HALF = 32
_E = 800000

_NC = 2
_NS = 16
_LANES = 16

_CHUNK = 80
_NCHUNKS = _E // _CHUNK
_CPT = _NCHUNKS // _NS
_NBUF = 6
_RPT = _NPAD // _NS
_RCHUNK = 56
_RSTEPS = _RPT // _RCHUNK


def _gcn_body(pk_hbm, tbl_hbm, out_hbm, lay_hbm,
              acc,
              pk0, pk1, pk2, pk3, pk4, pk5,
              r0_, r1_, r2_, r3_, r4_, r5_, zbuf, b0, b1, b2, b3,
              si0, si1, si2, si3, si4, si5,
              sg0, sg1, sg2, sg3, sg4, sg5,
              ss0, ss1, ss2, ss3, ss4, ss5):
  c = lax.axis_index("c")
  s = lax.axis_index("s")
  row_base = s * _RPT

  pk = (pk0, pk1, pk2, pk3, pk4, pk5)
  rows = (r0_, r1_, r2_, r3_, r4_, r5_)
  sem_i = (si0, si1, si2, si3, si4, si5)
  sem_g = (sg0, sg1, sg2, sg3, sg4, sg5)
  sem_s = (ss0, ss1, ss2, ss3, ss4, ss5)

  def _zb(i, _):
    zbuf[i, pl.ds(0, 16)] = jnp.zeros((16,), jnp.float32)
    zbuf[i, pl.ds(16, 16)] = jnp.zeros((16,), jnp.float32)
    return 0
  lax.fori_loop(0, _RCHUNK, _zb, 0, unroll=4)

  def zero_acc():
    def body(z, _):
      pltpu.sync_copy(zbuf, acc.at[pl.ds(row_base + z * _RCHUNK, _RCHUNK)])
      return 0
    lax.fori_loop(0, _RSTEPS, body, 0)

  def fetch_idx(j, b):
    r = s * _CPT + j
    pltpu.async_copy(pk_hbm.at[r], pk[b], sem_i[b])

  def start_gather(table, b):
    pltpu.make_async_copy(pk_hbm.at[0], pk[b], sem_i[b]).wait()
    pltpu.async_copy(table.at[pk[b].at[0]], rows[b], sem_g[b])

  def wait_scatter(b):
    pltpu.make_async_copy(rows[b], acc.at[pk[b].at[1]], sem_s[b]).wait()

  def process(table, b):
    pltpu.make_async_copy(table.at[pk[b].at[0]], rows[b], sem_g[b]).wait()

    @plsc.parallel_loop(0, _CHUNK // _LANES, unroll=5)
    def scale(g):
      wvec = plsc.bitcast(pk[b][2, pl.ds(g * _LANES, _LANES)], jnp.float32)
      base = g * _LANES
      for t in range(_LANES):
        we = jnp.full((_LANES,), wvec[t])
        e = base + t
        rows[b][e, pl.ds(0, 16)] = rows[b][e, pl.ds(0, 16)] * we
        rows[b][e, pl.ds(16, 16)] = rows[b][e, pl.ds(16, 16)] * we

    pltpu.async_copy(rows[b], acc.at[pk[b].at[1]], sem_s[b], add=True)

  def step(table, j, slot):
    b_cur = slot
    b_g = (slot + 2) % _NBUF
    b_f = (slot + 3) % _NBUF

    @pl.when(j >= 3)
    def _():
      wait_scatter(b_f)
    @pl.when(j + 3 < _CPT)
    def _():
      fetch_idx(j + 3, b_f)
    @pl.when(j + 2 < _CPT)
    def _():
      start_gather(table, b_g)
    process(table, b_cur)

  def run_layer(table):
    fetch_idx(0, 0)
    fetch_idx(1, 1)
    start_gather(table, 0)
    fetch_idx(2, 2)
    start_gather(table, 1)

    def hexa(k, _):
      for slot in range(_NBUF):
        step(table, _NBUF * k + slot, slot)
      return 0
    lax.fori_loop(0, _CPT // _NBUF, hexa, 0)

    step(table, _CPT - 1, 0)
    wait_scatter(4)
    wait_scatter(5)
    wait_scatter(0)

  def writeback(dst_table):
    def body(z, _):
      rr = row_base + z * _RCHUNK
      pltpu.sync_copy(acc.at[pl.ds(rr, _RCHUNK)],
                      dst_table.at[pl.ds(rr, _RCHUNK)])
      return 0
    lax.fori_loop(0, _RSTEPS, body, 0)

  zero_acc()
  plsc.subcore_barrier()
  run_layer(tbl_hbm.at[c])
  plsc.subcore_barrier()
  writeback(lay_hbm.at[0, c])
  plsc.subcore_barrier()

  zero_acc()
  plsc.subcore_barrier()
  run_layer(lay_hbm.at[0, c])
  plsc.subcore_barrier()
  writeback(lay_hbm.at[1, c])
  plsc.subcore_barrier()

  zero_acc()
  plsc.subcore_barrier()
  run_layer(lay_hbm.at[1, c])
  plsc.subcore_barrier()

  def mean_body(z, _):
    rr = row_base + z * _RCHUNK
    pltpu.sync_copy(tbl_hbm.at[c, pl.ds(rr, _RCHUNK)], b0)
    pltpu.sync_copy(lay_hbm.at[0, c, pl.ds(rr, _RCHUNK)], b1)
    pltpu.sync_copy(lay_hbm.at[1, c, pl.ds(rr, _RCHUNK)], b2)
    pltpu.sync_copy(acc.at[pl.ds(rr, _RCHUNK)], b3)

    def avg(i, _):
      for h in (0, 16):
        m = (b0[i, pl.ds(h, 16)] + b1[i, pl.ds(h, 16)]
             + b2[i, pl.ds(h, 16)] + b3[i, pl.ds(h, 16)]) * 0.25
        b0[i, pl.ds(h, 16)] = m
      return 0
    lax.fori_loop(0, _RCHUNK, avg, 0, unroll=4)

    pltpu.sync_copy(b0, out_hbm.at[c, pl.ds(rr, _RCHUNK)])
    return 0
  lax.fori_loop(0, _RSTEPS, mean_body, 0)


@jax.jit
def _gcn(pk, tbl):
  mesh = plsc.VectorSubcoreMesh(core_axis_name="c", subcore_axis_name="s",
                                num_cores=_NC, num_subcores=_NS)
  f = pl.kernel(
      _gcn_body,
      out_type=(
          jax.ShapeDtypeStruct((_NC, _NPAD, _HALF), jnp.float32),
          jax.ShapeDtypeStruct((2, _NC, _NPAD, _HALF), jnp.float32),
      ),
      mesh=mesh,
      compiler_params=pltpu.CompilerParams(use_tc_tiling_on_sc=False,
                                           needs_layout_passes=False),
      scratch_types=(
          [pltpu.VMEM_SHARED((_NPAD, _HALF), jnp.float32)]
          + [pltpu.VMEM((3, _CHUNK), jnp.int32)] * _NBUF
          + [pltpu.VMEM((_CHUNK, _HALF), jnp.float32)] * _NBUF
          + [pltpu.VMEM((_RCHUNK, _HALF), jnp.float32)] * 5
          + [pltpu.SemaphoreType.DMA] * (3 * _NBUF)
      ),
  )
  out, _ = f(pk, tbl)
  return out


def kernel(edge_index, edge_weight, user_emb, item_emb):
  src2d = edge_index[0].reshape(_NCHUNKS, _CHUNK)
  dst2d = edge_index[1].reshape(_NCHUNKS, _CHUNK)
  wbits = lax.bitcast_convert_type(edge_weight, jnp.int32).reshape(
      _NCHUNKS, _CHUNK)
  pk = jnp.stack([src2d, dst2d, wbits], axis=1)
  all_emb = jnp.concatenate([user_emb, item_emb], axis=0)
  pad = jnp.zeros((_NPAD - _N, _D), jnp.float32)
  all_emb = jnp.concatenate([all_emb, pad], axis=0)
  tbl = jnp.stack([all_emb[:, :_HALF], all_emb[:, _HALF:]], axis=0)
  out = _gcn(pk, tbl)
  final = jnp.concatenate([out[0, :_N], out[1, :_N]], axis=1)
  return final[:_NUM_USERS], final[_NUM_USERS:]
